# baseline (speedup 1.0000x reference)
import jax
import jax.numpy as jnp
from jax import lax
from jax.experimental import pallas as pl
from jax.experimental.pallas import tpu as pltpu

B = 16384
D = 1024
POS_WEIGHT = 2.0
EPS = 1e-8
K = 20
ROWS_PER_BLOCK = 1024
NUM_BLOCKS = B // ROWS_PER_BLOCK
SEL_ROWS = 128
SEL_COLS = B // SEL_ROWS
GROUPS = 8
GR = SEL_ROWS // GROUPS
BIG = 1 << 30
TILES = ROWS_PER_BLOCK // SEL_COLS


def _extract_groups(v, is_neg, lane):
    sent = -3.0 if is_neg else 3.0
    gflat = (lax.broadcasted_iota(jnp.int32, (GR, SEL_COLS), 0) * SEL_COLS
             + lax.broadcasted_iota(jnp.int32, (GR, SEL_COLS), 1))
    groups = [v[g * GR:(g + 1) * GR, :] for g in range(GROUPS)]
    rows = []
    for _ in range(K):
        acc = jnp.full((1, SEL_COLS), sent, jnp.float32)
        for g in range(GROUPS):
            vg = groups[g]
            m = jnp.max(vg) if is_neg else jnp.min(vg)
            sel = jnp.min(jnp.where(vg == m, gflat, BIG))
            groups[g] = jnp.where(gflat == sel, sent, vg)
            acc = jnp.where(lane == g, m, acc)
        rows.append(acc)
    return rows


def _tree(fn, xs):
    while len(xs) > 1:
        nxt = [fn(xs[2 * i], xs[2 * i + 1]) for i in range(len(xs) // 2)]
        if len(xs) % 2:
            nxt.append(xs[-1])
        xs = nxt
    return xs[0]


def _merge_rows(rows, is_neg, lane):
    sent = -3.0 if is_neg else 3.0
    lab = 0.0 if is_neg else 1.0
    flats = [lane + r * SEL_COLS for r in range(K)]
    tot = jnp.float32(0.0)
    for _ in range(K):
        red = _tree(jnp.maximum if is_neg else jnp.minimum, rows)
        m = jnp.max(red) if is_neg else jnp.min(red)
        sel = jnp.min(_tree(jnp.minimum,
                            [jnp.where(rows[r] == m, flats[r], BIG)
                             for r in range(K)]))
        rows = [jnp.where(flats[r] == sel, sent, rows[r]) for r in range(K)]
        d = m - lab
        tot = tot + d * d
    return tot


def _fused_kernel(a_ref, b_ref, lab_ref, o_ref, sim_acc):
    i = pl.program_id(0)
    a = a_ref[...]
    b = b_ref[...]
    num = jnp.sum(a * b, axis=1, keepdims=True)
    na = jnp.sqrt(jnp.sum(a * a, axis=1, keepdims=True))
    nb = jnp.sqrt(jnp.sum(b * b, axis=1, keepdims=True))
    sim = num / jnp.maximum(na * nb, EPS)
    sim_acc[pl.ds(TILES * i, TILES), :] = sim.reshape(TILES, SEL_COLS)

    @pl.when(i == NUM_BLOCKS - 1)
    def _select():
        simf = sim_acc[...]
        labf = lab_ref[...]
        neg = jnp.where(labf == 0.0, simf, -3.0)
        pos = jnp.where(labf == 0.0, 3.0, simf)
        lane = lax.broadcasted_iota(jnp.int32, (1, SEL_COLS), 1)
        nrows = _extract_groups(neg, True, lane)
        prows = _extract_groups(pos, False, lane)
        tot_n = _merge_rows(nrows, True, lane)
        tot_p = _merge_rows(prows, False, lane)
        o_ref[...] = jnp.broadcast_to((tot_n + tot_p) * (1.0 / (2 * K)), (1, 1))


def kernel(sample_1, sample_2, labels, original_target):
    lab2d = labels.reshape(SEL_ROWS, SEL_COLS)
    out = pl.pallas_call(
        _fused_kernel,
        grid=(NUM_BLOCKS,),
        in_specs=[
            pl.BlockSpec((ROWS_PER_BLOCK, D), lambda i: (i, 0)),
            pl.BlockSpec((ROWS_PER_BLOCK, D), lambda i: (i, 0)),
            pl.BlockSpec((SEL_ROWS, SEL_COLS), lambda i: (0, 0)),
        ],
        out_specs=pl.BlockSpec((1, 1), lambda i: (0, 0)),
        out_shape=jax.ShapeDtypeStruct((1, 1), jnp.float32),
        scratch_shapes=[pltpu.VMEM((SEL_ROWS, SEL_COLS), jnp.float32)],
    )(sample_1, sample_2, lab2d)

    weight = (POS_WEIGHT - 1.0) * jnp.float32(original_target) + 1.0
    return out[0, 0] * weight


# speedup vs baseline: 2.4624x; 2.4624x over previous
import jax
import jax.numpy as jnp
from jax import lax
from jax.experimental import pallas as pl
from jax.experimental.pallas import tpu as pltpu

B = 16384
D = 1024
POS_WEIGHT = 2.0
EPS = 1e-8
K = 20
ROWS_PER_BLOCK = 1024
NUM_BLOCKS = B // ROWS_PER_BLOCK
SEL_ROWS = 128
SEL_COLS = B // SEL_ROWS
BIG = 1 << 30
TILES = ROWS_PER_BLOCK // SEL_COLS


def _fused_kernel(a_ref, b_ref, lab_ref, o_ref, sim_acc):
    i = pl.program_id(0)
    a = a_ref[...]
    b = b_ref[...]
    num = jnp.sum(a * b, axis=1, keepdims=True)
    na = jnp.sqrt(jnp.sum(a * a, axis=1, keepdims=True))
    nb = jnp.sqrt(jnp.sum(b * b, axis=1, keepdims=True))
    sim = num / jnp.maximum(na * nb, EPS)
    sim_acc[pl.ds(TILES * i, TILES), :] = sim.reshape(TILES, SEL_COLS)

    @pl.when(i == NUM_BLOCKS - 1)
    def _select():
        simf = sim_acc[...]
        lab = lab_ref[...]
        neg = jnp.where(lab == 0.0, simf, -3.0)
        pos = jnp.where(lab == 0.0, 3.0, simf)
        r = lax.broadcasted_iota(jnp.int32, (SEL_ROWS, SEL_COLS), 0)
        c = lax.broadcasted_iota(jnp.int32, (SEL_ROWS, SEL_COLS), 1)
        flat = r * SEL_COLS + c

        vn, vp = neg, pos
        tot_n = jnp.float32(0.0)
        tot_p = jnp.float32(0.0)
        rem_n = jnp.float32(K)
        rem_p = jnp.float32(K)
        one = jnp.ones((SEL_ROWS, SEL_COLS), jnp.float32)
        zero = jnp.zeros((SEL_ROWS, SEL_COLS), jnp.float32)
        for _ in range(K):
            mn = jnp.max(vn)
            mp = jnp.min(vp)
            eq_n = vn == mn
            eq_p = vp == mp
            vn = jnp.where(eq_n, -3.0, vn)
            vp = jnp.where(eq_p, 3.0, vp)
            cnt_n = jnp.sum(jnp.where(eq_n, one, zero))
            cnt_p = jnp.sum(jnp.where(eq_p, one, zero))
            take_n = jnp.maximum(jnp.minimum(cnt_n, rem_n), 0.0)
            take_p = jnp.maximum(jnp.minimum(cnt_p, rem_p), 0.0)
            rem_n = rem_n - cnt_n
            rem_p = rem_p - cnt_p
            dp = mp - 1.0
            tot_n = tot_n + take_n * mn * mn
            tot_p = tot_p + take_p * dp * dp
        o_ref[...] = jnp.broadcast_to((tot_n + tot_p) * (1.0 / (2 * K)), (1, 1))


def kernel(sample_1, sample_2, labels, original_target):
    lab2d = labels.reshape(SEL_ROWS, SEL_COLS)
    out = pl.pallas_call(
        _fused_kernel,
        grid=(NUM_BLOCKS,),
        in_specs=[
            pl.BlockSpec((ROWS_PER_BLOCK, D), lambda i: (i, 0)),
            pl.BlockSpec((ROWS_PER_BLOCK, D), lambda i: (i, 0)),
            pl.BlockSpec((SEL_ROWS, SEL_COLS), lambda i: (0, 0)),
        ],
        out_specs=pl.BlockSpec((1, 1), lambda i: (0, 0)),
        out_shape=jax.ShapeDtypeStruct((1, 1), jnp.float32),
        scratch_shapes=[pltpu.VMEM((SEL_ROWS, SEL_COLS), jnp.float32)],
    )(sample_1, sample_2, lab2d)

    weight = (POS_WEIGHT - 1.0) * jnp.float32(original_target) + 1.0
    return out[0, 0] * weight
